# baseline (speedup 1.0000x reference)
import functools

import jax
import jax.numpy as jnp
from jax.experimental import pallas as pl
from jax.experimental.pallas import tpu as pltpu

_N = 16384
_C = 1000
_GAMMA = 2.0
_B = 2048


def _focal_body(x_ref, t_ref, a_ref, out_ref):
    i = pl.program_id(0)
    nb = pl.num_programs(0)
    x = x_ref[...]
    t = t_ref[0, 0, :]
    a = a_ref[...]

    m = jnp.max(x, axis=0)
    e = jnp.exp(x - m[None, :])
    s = jnp.sum(e, axis=0)

    iota = jax.lax.broadcasted_iota(jnp.int32, x.shape, 0)
    onehot = iota == t[None, :]
    xt = jnp.sum(jnp.where(onehot, x, 0.0), axis=0)
    at = jnp.sum(jnp.where(onehot, a, 0.0), axis=0)

    logp = (xt - m) - jnp.log(s)
    p = jnp.exp(xt - m) / s
    omp = 1.0 - p
    loss = -at * (omp * omp) * logp
    bsum = jnp.sum(loss, keepdims=True).reshape(1, 1)

    @pl.when(i == 0)
    def _init():
        out_ref[...] = jnp.zeros((1, 1), jnp.float32)

    acc = out_ref[...] + bsum
    out_ref[...] = jnp.where(i == nb - 1, acc * (1.0 / _N), acc)


@jax.jit
def kernel(inputs, targets, alpha):
    nb = _N // _B
    xt_view = inputs.T
    t3 = targets.reshape(nb, 1, _B)
    out = pl.pallas_call(
        _focal_body,
        grid=(nb,),
        in_specs=[
            pl.BlockSpec((_C, _B), lambda i: (0, i)),
            pl.BlockSpec((1, 1, _B), lambda i: (i, 0, 0)),
            pl.BlockSpec((_C, 1), lambda i: (0, 0)),
        ],
        out_specs=pl.BlockSpec((1, 1), lambda i: (0, 0)),
        out_shape=jax.ShapeDtypeStruct((1, 1), jnp.float32),
    )(xt_view, t3, alpha)
    return out[0, 0]


# speedup vs baseline: 1.1288x; 1.1288x over previous
import functools

import jax
import jax.numpy as jnp
from jax.experimental import pallas as pl
from jax.experimental.pallas import tpu as pltpu

_N = 16384
_C = 1000
_GAMMA = 2.0
_B = 1024


def _focal_body(x_ref, t_ref, a_ref, out_ref):
    i = pl.program_id(0)
    nb = pl.num_programs(0)
    x = x_ref[...]
    t = t_ref[0, 0, :]
    a = a_ref[...]

    m = jnp.max(x, axis=0)
    e = jnp.exp(x - m[None, :])

    iota = jax.lax.broadcasted_iota(jnp.int32, x.shape, 0)
    onehot = iota == t[None, :]
    em = jnp.where(onehot, e, 0.0)
    am = jnp.where(onehot, a, 0.0)

    ones = jnp.ones((1, _C), jnp.float32)
    s = jax.lax.dot_general(ones, e, (((1,), (0,)), ((), ())),
                            preferred_element_type=jnp.float32)
    pe = jax.lax.dot_general(ones, em, (((1,), (0,)), ((), ())),
                             preferred_element_type=jnp.float32)
    at = jax.lax.dot_general(ones, am, (((1,), (0,)), ((), ())),
                             preferred_element_type=jnp.float32)

    p = pe / s
    logp = jnp.log(p)
    omp = 1.0 - p
    loss = -at * (omp * omp) * logp
    bsum = jnp.sum(loss, keepdims=True).reshape(1, 1)

    @pl.when(i == 0)
    def _init():
        out_ref[...] = jnp.zeros((1, 1), jnp.float32)

    acc = out_ref[...] + bsum
    out_ref[...] = jnp.where(i == nb - 1, acc * (1.0 / _N), acc)


@jax.jit
def kernel(inputs, targets, alpha):
    nb = _N // _B
    xt_view = inputs.T
    t3 = targets.reshape(nb, 1, _B)
    out = pl.pallas_call(
        _focal_body,
        grid=(nb,),
        in_specs=[
            pl.BlockSpec((_C, _B), lambda i: (0, i)),
            pl.BlockSpec((1, 1, _B), lambda i: (i, 0, 0)),
            pl.BlockSpec((_C, 1), lambda i: (0, 0)),
        ],
        out_specs=pl.BlockSpec((1, 1), lambda i: (0, 0)),
        out_shape=jax.ShapeDtypeStruct((1, 1), jnp.float32),
    )(xt_view, t3, alpha)
    return out[0, 0]
